# baseline (speedup 1.0000x reference)
import functools

import jax
import jax.numpy as jnp
import numpy as np
from jax.experimental import pallas as pl
from jax.experimental.pallas import tpu as pltpu

_B, _H, _W = 8, 512, 512
_GH, _GW, _GD = 16, 16, 8
_NC = 12
_NIN = 3
_NOUT = 3
_RB = 16
_NBLK = _H // _RB

_GP = _GW + 2
_ROWS = _GP * _NC * _GD


def _bxt_const() -> np.ndarray:
    w = np.arange(_W, dtype=np.int64)
    gx = (w + 0.5) * _GW / _W - 0.5
    x0 = np.floor(gx).astype(np.int64)
    fx = gx - x0
    bxt = np.zeros((_GP, _W), dtype=np.float64)
    bxt[x0 + 1, w] = 1.0 - fx
    bxt[x0 + 2, w] = fx
    return bxt.astype(np.float32)


def _slice_kernel(lhs_ref, bxt_ref, guide_ref, inp_ref, out_ref):
    j = pl.program_id(1)

    ya = (j + 1) // 2
    gab = jnp.dot(
        lhs_ref[0, pl.ds(ya * _NC * _GD, 2 * _NC * _GD), :],
        bxt_ref[...],
        preferred_element_type=jnp.float32,
    )
    ga = gab[: _NC * _GD].reshape(_NC, _GD, _W)
    gb = gab[_NC * _GD :].reshape(_NC, _GD, _W)

    r = jax.lax.broadcasted_iota(jnp.int32, (_RB, 1), 0).astype(jnp.float32)
    hrow = j.astype(jnp.float32) * _RB + r
    gy = (hrow + 0.5) * (_GH / _H) - 0.5
    wy = gy - (ya.astype(jnp.float32) - 1.0)

    g = guide_ref[0]
    gz = g * _GD - 0.5
    z0 = jnp.floor(gz)
    wz = gz - z0
    z0i = z0.astype(jnp.int32)
    z0c = jnp.clip(z0i, 0, _GD - 1)
    z1c = jnp.clip(z0i + 1, 0, _GD - 1)

    wya = 1.0 - wy
    wa = []
    wb = []
    for z in range(_GD):
        wzz = jnp.where(z0c == z, 1.0 - wz, 0.0) + jnp.where(z1c == z, wz, 0.0)
        wa.append(wzz * wya)
        wb.append(wzz * wy)

    x0p = inp_ref[0, 0]
    x1p = inp_ref[0, 1]
    x2p = inp_ref[0, 2]
    xs = (x0p, x1p, x2p)

    for o in range(_NOUT):
        acc = None
        for i in range(_NIN + 1):
            c = o * (_NIN + 1) + i
            cacc = wa[0] * ga[c, 0][None, :] + wb[0] * gb[c, 0][None, :]
            for z in range(1, _GD):
                cacc = cacc + wa[z] * ga[c, z][None, :] + wb[z] * gb[c, z][None, :]
            if i < _NIN:
                term = cacc * xs[i]
            else:
                term = cacc
            acc = term if acc is None else acc + term
        out_ref[0, o] = acc


@jax.jit
def kernel(bilateral_grid, guide, input):
    bgrid, inp = bilateral_grid, input
    gridp = jnp.pad(bgrid, ((0, 0), (1, 1), (1, 1), (0, 0), (0, 0)), mode="edge")
    lhs = gridp.transpose(0, 1, 4, 3, 2).reshape(_B, _ROWS, _GP)
    bxt = jnp.asarray(_bxt_const())
    inp_t = inp.transpose(0, 3, 1, 2)

    out_t = pl.pallas_call(
        _slice_kernel,
        grid=(_B, _NBLK),
        in_specs=[
            pl.BlockSpec((1, _ROWS, _GP), lambda b, j: (b, 0, 0)),
            pl.BlockSpec((_GP, _W), lambda b, j: (0, 0)),
            pl.BlockSpec((1, _RB, _W), lambda b, j: (b, j, 0)),
            pl.BlockSpec((1, _NIN, _RB, _W), lambda b, j: (b, 0, j, 0)),
        ],
        out_specs=pl.BlockSpec((1, _NOUT, _RB, _W), lambda b, j: (b, 0, j, 0)),
        out_shape=jax.ShapeDtypeStruct((_B, _NOUT, _H, _W), jnp.float32),
    )(lhs, bxt, guide, inp_t)

    return out_t.transpose(0, 2, 3, 1)


# speedup vs baseline: 1.0000x; 1.0000x over previous
import functools

import jax
import jax.numpy as jnp
import numpy as np
from jax.experimental import pallas as pl
from jax.experimental.pallas import tpu as pltpu

_B, _H, _W = 8, 512, 512
_GH, _GW, _GD = 16, 16, 8
_NC = 12
_NIN = 3
_NOUT = 3
_RB = 16
_NBLK = _H // _RB

_GP = _GW + 2
_ROWS = _GP * _NC * _GD


def _bxt_const() -> np.ndarray:
    w = np.arange(_W, dtype=np.int64)
    gx = (w + 0.5) * _GW / _W - 0.5
    x0 = np.floor(gx).astype(np.int64)
    fx = gx - x0
    bxt = np.zeros((_GP, _W), dtype=np.float64)
    bxt[x0 + 1, w] = 1.0 - fx
    bxt[x0 + 2, w] = fx
    return bxt.astype(np.float32)


def _slice_kernel(lhs_ref, bxt_ref, guide_ref, inp_ref, out_ref, gx_scr):
    j = pl.program_id(1)

    ya = (j + 1) // 2
    gx_scr[...] = jnp.dot(
        lhs_ref[0, pl.ds(ya * _NC * _GD, 2 * _NC * _GD), :],
        bxt_ref[...],
        preferred_element_type=jnp.float32,
    )
    gab = gx_scr[...]
    ga = gab[: _NC * _GD].reshape(_NC, _GD, _W)
    gb = gab[_NC * _GD :].reshape(_NC, _GD, _W)

    r = jax.lax.broadcasted_iota(jnp.int32, (_RB, 1), 0).astype(jnp.float32)
    hrow = j.astype(jnp.float32) * _RB + r
    gy = (hrow + 0.5) * (_GH / _H) - 0.5
    wy = gy - (ya.astype(jnp.float32) - 1.0)

    g = guide_ref[0]
    gz = g * _GD - 0.5
    z0 = jnp.floor(gz)
    wz = gz - z0
    z0i = z0.astype(jnp.int32)
    z0c = jnp.clip(z0i, 0, _GD - 1)
    z1c = jnp.clip(z0i + 1, 0, _GD - 1)

    wya = 1.0 - wy
    wa = []
    wb = []
    for z in range(_GD):
        wzz = jnp.where(z0c == z, 1.0 - wz, 0.0) + jnp.where(z1c == z, wz, 0.0)
        wa.append(wzz * wya)
        wb.append(wzz * wy)

    x0p = inp_ref[0, 0]
    x1p = inp_ref[0, 1]
    x2p = inp_ref[0, 2]
    xs = (x0p, x1p, x2p)

    for o in range(_NOUT):
        acc = None
        for i in range(_NIN + 1):
            c = o * (_NIN + 1) + i
            cacc = wa[0] * ga[c, 0][None, :] + wb[0] * gb[c, 0][None, :]
            for z in range(1, _GD):
                cacc = cacc + wa[z] * ga[c, z][None, :] + wb[z] * gb[c, z][None, :]
            if i < _NIN:
                term = cacc * xs[i]
            else:
                term = cacc
            acc = term if acc is None else acc + term
        out_ref[0, o] = acc


@jax.jit
def kernel(bilateral_grid, guide, input):
    bgrid, inp = bilateral_grid, input
    gridp = jnp.pad(bgrid, ((0, 0), (1, 1), (1, 1), (0, 0), (0, 0)), mode="edge")
    lhs = gridp.transpose(0, 1, 4, 3, 2).reshape(_B, _ROWS, _GP)
    bxt = jnp.asarray(_bxt_const())
    inp_t = inp.transpose(0, 3, 1, 2)

    out_t = pl.pallas_call(
        _slice_kernel,
        grid=(_B, _NBLK),
        in_specs=[
            pl.BlockSpec((1, _ROWS, _GP), lambda b, j: (b, 0, 0)),
            pl.BlockSpec((_GP, _W), lambda b, j: (0, 0)),
            pl.BlockSpec((1, _RB, _W), lambda b, j: (b, j, 0)),
            pl.BlockSpec((1, _NIN, _RB, _W), lambda b, j: (b, 0, j, 0)),
        ],
        out_specs=pl.BlockSpec((1, _NOUT, _RB, _W), lambda b, j: (b, 0, j, 0)),
        out_shape=jax.ShapeDtypeStruct((_B, _NOUT, _H, _W), jnp.float32),
        scratch_shapes=[pltpu.VMEM((2 * _NC * _GD, _W), jnp.float32)],
    )(lhs, bxt, guide, inp_t)

    return out_t.transpose(0, 2, 3, 1)


# speedup vs baseline: 1.3313x; 1.3313x over previous
import jax
import jax.numpy as jnp
import numpy as np
from jax.experimental import pallas as pl
from jax.experimental.pallas import tpu as pltpu

_B, _H, _W = 8, 512, 512
_GH, _GW, _GD = 16, 16, 8
_NC = 12
_NIN = 3
_NOUT = 3

_NH = 4
_RB = 16 * _NH
_NBLK = _H // _RB
_GRP = _NC * _GD
_NGRP = _NH // 2 + 2

_GP = _GW + 2
_ROWS = _GP * _GRP


def _bxt_const() -> np.ndarray:
    w = np.arange(_W, dtype=np.int64)
    gx = (w + 0.5) * _GW / _W - 0.5
    x0 = np.floor(gx).astype(np.int64)
    fx = gx - x0
    bxt = np.zeros((_GP, _W), dtype=np.float64)
    bxt[x0 + 1, w] = 1.0 - fx
    bxt[x0 + 2, w] = fx
    return bxt.astype(np.float32)


def _slice_kernel(lhs_ref, bxt_ref, guide_ref, inp_ref, out_ref, gx_scr):
    j = pl.program_id(1)

    g0 = (_NH // 2) * j
    gx_scr[...] = jnp.dot(
        lhs_ref[0, pl.ds(g0 * _GRP, _NGRP * _GRP), :],
        bxt_ref[...],
        preferred_element_type=jnp.float32,
    )

    for k in range(_NH):
        m = _NH * j + k
        ya = (m + 1) // 2
        loc = (k + 1) // 2
        gab = gx_scr[loc * _GRP : (loc + 2) * _GRP]
        ga = gab[:_GRP].reshape(_NC, _GD, _W)
        gb = gab[_GRP:].reshape(_NC, _GD, _W)

        r = jax.lax.broadcasted_iota(jnp.int32, (16, 1), 0).astype(jnp.float32)
        hrow = j.astype(jnp.float32) * _RB + (16 * k) + r
        gy = (hrow + 0.5) * (_GH / _H) - 0.5
        wy = gy - (ya.astype(jnp.float32) - 1.0)

        g = guide_ref[0, 16 * k : 16 * (k + 1)]
        gz = g * _GD - 0.5
        z0 = jnp.floor(gz)
        wz = gz - z0
        z0i = z0.astype(jnp.int32)
        z0c = jnp.clip(z0i, 0, _GD - 1)
        z1c = jnp.clip(z0i + 1, 0, _GD - 1)

        wya = 1.0 - wy
        wa = []
        wb = []
        for z in range(_GD):
            wzz = jnp.where(z0c == z, 1.0 - wz, 0.0) + jnp.where(z1c == z, wz, 0.0)
            wa.append(wzz * wya)
            wb.append(wzz * wy)

        xs = tuple(inp_ref[0, i, 16 * k : 16 * (k + 1)] for i in range(_NIN))

        for o in range(_NOUT):
            acc = None
            for i in range(_NIN + 1):
                c = o * (_NIN + 1) + i
                cacc = wa[0] * ga[c, 0][None, :] + wb[0] * gb[c, 0][None, :]
                for z in range(1, _GD):
                    cacc = cacc + wa[z] * ga[c, z][None, :] + wb[z] * gb[c, z][None, :]
                if i < _NIN:
                    term = cacc * xs[i]
                else:
                    term = cacc
                acc = term if acc is None else acc + term
            out_ref[0, o, 16 * k : 16 * (k + 1)] = acc


@jax.jit
def kernel(bilateral_grid, guide, input):
    bgrid, inp = bilateral_grid, input
    gridp = jnp.pad(bgrid, ((0, 0), (1, 1), (1, 1), (0, 0), (0, 0)), mode="edge")
    lhs = gridp.transpose(0, 1, 4, 3, 2).reshape(_B, _ROWS, _GP)
    bxt = jnp.asarray(_bxt_const())
    inp_t = inp.transpose(0, 3, 1, 2)

    out_t = pl.pallas_call(
        _slice_kernel,
        grid=(_B, _NBLK),
        in_specs=[
            pl.BlockSpec((1, _ROWS, _GP), lambda b, j: (b, 0, 0)),
            pl.BlockSpec((_GP, _W), lambda b, j: (0, 0)),
            pl.BlockSpec((1, _RB, _W), lambda b, j: (b, j, 0)),
            pl.BlockSpec((1, _NIN, _RB, _W), lambda b, j: (b, 0, j, 0)),
        ],
        out_specs=pl.BlockSpec((1, _NOUT, _RB, _W), lambda b, j: (b, 0, j, 0)),
        out_shape=jax.ShapeDtypeStruct((_B, _NOUT, _H, _W), jnp.float32),
        scratch_shapes=[pltpu.VMEM((_NGRP * _GRP, _W), jnp.float32)],
    )(lhs, bxt, guide, inp_t)

    return out_t.transpose(0, 2, 3, 1)


# speedup vs baseline: 1.4024x; 1.0534x over previous
import jax
import jax.numpy as jnp
import numpy as np
from jax.experimental import pallas as pl
from jax.experimental.pallas import tpu as pltpu

_B, _H, _W = 8, 512, 512
_GH, _GW, _GD = 16, 16, 8
_NC = 12
_NIN = 3
_NOUT = 3

_NH = 8
_RB = 16 * _NH
_NBLK = _H // _RB
_GRP = _NC * _GD
_NGRP = _NH // 2 + 2

_GP = _GW + 2
_ROWS = _GP * _GRP


def _bxt_const() -> np.ndarray:
    w = np.arange(_W, dtype=np.int64)
    gx = (w + 0.5) * _GW / _W - 0.5
    x0 = np.floor(gx).astype(np.int64)
    fx = gx - x0
    bxt = np.zeros((_GP, _W), dtype=np.float64)
    bxt[x0 + 1, w] = 1.0 - fx
    bxt[x0 + 2, w] = fx
    return bxt.astype(np.float32)


def _slice_kernel(lhs_ref, bxt_ref, guide_ref, inp_ref, out_ref, gx_scr):
    j = pl.program_id(1)

    g0 = (_NH // 2) * j
    gx_scr[...] = jnp.dot(
        lhs_ref[0, pl.ds(g0 * _GRP, _NGRP * _GRP), :],
        bxt_ref[...],
        preferred_element_type=jnp.float32,
    )

    for k in range(_NH):
        m = _NH * j + k
        ya = (m + 1) // 2
        loc = (k + 1) // 2
        gab = gx_scr[loc * _GRP : (loc + 2) * _GRP]
        ga = gab[:_GRP].reshape(_NC, _GD, _W)
        gb = gab[_GRP:].reshape(_NC, _GD, _W)

        r = jax.lax.broadcasted_iota(jnp.int32, (16, 1), 0).astype(jnp.float32)
        hrow = j.astype(jnp.float32) * _RB + (16 * k) + r
        gy = (hrow + 0.5) * (_GH / _H) - 0.5
        wy = gy - (ya.astype(jnp.float32) - 1.0)

        g = guide_ref[0, 16 * k : 16 * (k + 1)]
        gz = g * _GD - 0.5
        z0 = jnp.floor(gz)
        wz = gz - z0
        z0i = z0.astype(jnp.int32)
        z0c = jnp.clip(z0i, 0, _GD - 1)
        z1c = jnp.clip(z0i + 1, 0, _GD - 1)

        wya = 1.0 - wy
        wa = []
        wb = []
        for z in range(_GD):
            wzz = jnp.where(z0c == z, 1.0 - wz, 0.0) + jnp.where(z1c == z, wz, 0.0)
            wa.append(wzz * wya)
            wb.append(wzz * wy)

        xs = tuple(inp_ref[0, i, 16 * k : 16 * (k + 1)] for i in range(_NIN))

        for o in range(_NOUT):
            acc = None
            for i in range(_NIN + 1):
                c = o * (_NIN + 1) + i
                cacc = wa[0] * ga[c, 0][None, :] + wb[0] * gb[c, 0][None, :]
                for z in range(1, _GD):
                    cacc = cacc + wa[z] * ga[c, z][None, :] + wb[z] * gb[c, z][None, :]
                if i < _NIN:
                    term = cacc * xs[i]
                else:
                    term = cacc
                acc = term if acc is None else acc + term
            out_ref[0, o, 16 * k : 16 * (k + 1)] = acc


@jax.jit
def kernel(bilateral_grid, guide, input):
    bgrid, inp = bilateral_grid, input
    gridp = jnp.pad(bgrid, ((0, 0), (1, 1), (1, 1), (0, 0), (0, 0)), mode="edge")
    lhs = gridp.transpose(0, 1, 4, 3, 2).reshape(_B, _ROWS, _GP)
    bxt = jnp.asarray(_bxt_const())
    inp_t = inp.transpose(0, 3, 1, 2)

    out_t = pl.pallas_call(
        _slice_kernel,
        grid=(_B, _NBLK),
        in_specs=[
            pl.BlockSpec((1, _ROWS, _GP), lambda b, j: (b, 0, 0)),
            pl.BlockSpec((_GP, _W), lambda b, j: (0, 0)),
            pl.BlockSpec((1, _RB, _W), lambda b, j: (b, j, 0)),
            pl.BlockSpec((1, _NIN, _RB, _W), lambda b, j: (b, 0, j, 0)),
        ],
        out_specs=pl.BlockSpec((1, _NOUT, _RB, _W), lambda b, j: (b, 0, j, 0)),
        out_shape=jax.ShapeDtypeStruct((_B, _NOUT, _H, _W), jnp.float32),
        scratch_shapes=[pltpu.VMEM((_NGRP * _GRP, _W), jnp.float32)],
    )(lhs, bxt, guide, inp_t)

    return out_t.transpose(0, 2, 3, 1)
